# baseline (speedup 1.0000x reference)
import functools

import jax
import jax.numpy as jnp
from jax.experimental import pallas as pl
from jax.experimental.pallas import tpu as pltpu

H, W, C = 64, 64, 21
N = H * W
THETA_ALPHA, THETA_BETA, THETA_GAMMA = 80.0, 13.0, 3.0
BILATERAL_COMPAT, SPATIAL_COMPAT = 10.0, 3.0
NUM_ITERATIONS = 5

CP = 128
BIG = 1.0e9

RT_A = 256
BT = 512
NI = N // BT
NT_A = N // RT_A


def _norms_body(fb_ref, fbT_ref, fs_ref, fsT_ref, nb_ref, ns_ref):
    for f_ref, fT_ref, out_ref in ((fb_ref, fbT_ref, nb_ref),
                                   (fs_ref, fsT_ref, ns_ref)):
        fi = f_ref[...]
        fT = fT_ref[...]
        sqi = jnp.sum(fi * fi, axis=1, keepdims=True)
        sqj = jnp.sum(fT * fT, axis=0, keepdims=True)
        cross = jnp.dot(fi, fT, preferred_element_type=jnp.float32)
        d2 = jnp.maximum(sqi + sqj - 2.0 * cross, 0.0)
        k = jnp.exp(-0.5 * d2)
        rs = jnp.sum(k, axis=1, keepdims=True)
        out_ref[...] = 1.0 / (jnp.sqrt(rs) + 1e-20)


def _build_m_body(fb_ref, fbT_ref, fs_ref, fsT_ref,
                  nb_ref, nbT_ref, ns_ref, nsT_ref, m_ref):
    def tile_kernel(f_ref, fT_ref):
        fi = f_ref[...]
        fT = fT_ref[...]
        sqi = jnp.sum(fi * fi, axis=1, keepdims=True)
        sqj = jnp.sum(fT * fT, axis=0, keepdims=True)
        cross = jnp.dot(fi, fT, preferred_element_type=jnp.float32)
        d2 = jnp.maximum(sqi + sqj - 2.0 * cross, 0.0)
        return jnp.exp(-0.5 * d2)

    kb = tile_kernel(fb_ref, fbT_ref)
    ks = tile_kernel(fs_ref, fsT_ref)
    m_ref[...] = (BILATERAL_COMPAT * (nb_ref[...] * nbT_ref[...]) * kb
                  + SPATIAL_COMPAT * (ns_ref[...] * nsT_ref[...]) * ks)


def _softmax(x):
    m = jnp.max(x, axis=-1, keepdims=True)
    e = jnp.exp(x - m)
    return e / jnp.sum(e, axis=-1, keepdims=True)


def _iterate_body(u_ref, m_ref, out_ref, qa, qb, acc):
    t = pl.program_id(0)
    i = pl.program_id(1)
    j = pl.program_id(2)

    @pl.when((t == 0) & (i == 0) & (j == 0))
    def _init():
        qa[...] = _softmax(-u_ref[...])

    @pl.when(j == 0)
    def _zero():
        acc[...] = jnp.zeros_like(acc)

    read_a = (t % 2) == 0
    m = m_ref[...]

    @pl.when(read_a)
    def _dot_a():
        acc[...] += jnp.dot(m, qa[pl.ds(j * BT, BT), :],
                            preferred_element_type=jnp.float32)

    @pl.when(jnp.logical_not(read_a))
    def _dot_b():
        acc[...] += jnp.dot(m, qb[pl.ds(j * BT, BT), :],
                            preferred_element_type=jnp.float32)

    @pl.when(j == NI - 1)
    def _finish():
        logits = acc[...] - u_ref[pl.ds(i * BT, BT), :]
        qnew = _softmax(logits)

        @pl.when(read_a)
        def _wb():
            qb[pl.ds(i * BT, BT), :] = qnew

        @pl.when(jnp.logical_not(read_a))
        def _wa():
            qa[pl.ds(i * BT, BT), :] = qnew

        @pl.when(t == NUM_ITERATIONS - 1)
        def _out():
            out_ref[pl.ds(i * BT, BT), :] = qnew


@jax.jit
def kernel(unary, image):
    f32 = jnp.float32
    ys, xs = jnp.meshgrid(jnp.arange(H, dtype=f32),
                          jnp.arange(W, dtype=f32), indexing="ij")
    zeros1 = jnp.zeros((N, 1), f32)
    fb = jnp.concatenate(
        [(xs / THETA_ALPHA).reshape(N, 1), (ys / THETA_ALPHA).reshape(N, 1),
         (image / THETA_BETA).reshape(N, 3), zeros1, zeros1, zeros1], axis=1)
    fs = jnp.concatenate(
        [(xs / THETA_GAMMA).reshape(N, 1), (ys / THETA_GAMMA).reshape(N, 1)]
        + [zeros1] * 6, axis=1)
    fbT = fb.T
    fsT = fs.T

    nb, ns = pl.pallas_call(
        _norms_body,
        grid=(NT_A,),
        in_specs=[
            pl.BlockSpec((RT_A, 8), lambda i: (i, 0)),
            pl.BlockSpec((8, N), lambda i: (0, 0)),
            pl.BlockSpec((RT_A, 8), lambda i: (i, 0)),
            pl.BlockSpec((8, N), lambda i: (0, 0)),
        ],
        out_specs=[
            pl.BlockSpec((RT_A, 1), lambda i: (i, 0)),
            pl.BlockSpec((RT_A, 1), lambda i: (i, 0)),
        ],
        out_shape=[
            jax.ShapeDtypeStruct((N, 1), f32),
            jax.ShapeDtypeStruct((N, 1), f32),
        ],
    )(fb, fbT, fs, fsT)
    nbT = nb.reshape(1, N)
    nsT = ns.reshape(1, N)

    m = pl.pallas_call(
        _build_m_body,
        grid=(NI, NI),
        in_specs=[
            pl.BlockSpec((BT, 8), lambda i, j: (i, 0)),
            pl.BlockSpec((8, BT), lambda i, j: (0, j)),
            pl.BlockSpec((BT, 8), lambda i, j: (i, 0)),
            pl.BlockSpec((8, BT), lambda i, j: (0, j)),
            pl.BlockSpec((BT, 1), lambda i, j: (i, 0)),
            pl.BlockSpec((1, BT), lambda i, j: (0, j)),
            pl.BlockSpec((BT, 1), lambda i, j: (i, 0)),
            pl.BlockSpec((1, BT), lambda i, j: (0, j)),
        ],
        out_specs=pl.BlockSpec((BT, BT), lambda i, j: (i, j)),
        out_shape=jax.ShapeDtypeStruct((N, N), f32),
    )(fb, fbT, fs, fsT, nb, nbT, ns, nsT)

    u = unary.reshape(N, C)
    u_pad = jnp.full((N, CP), BIG, f32).at[:, :C].set(u)

    q = pl.pallas_call(
        _iterate_body,
        grid=(NUM_ITERATIONS, NI, NI),
        in_specs=[
            pl.BlockSpec((N, CP), lambda t, i, j: (0, 0)),
            pl.BlockSpec((BT, BT), lambda t, i, j: (i, j)),
        ],
        out_specs=pl.BlockSpec((N, CP), lambda t, i, j: (0, 0)),
        out_shape=jax.ShapeDtypeStruct((N, CP), f32),
        scratch_shapes=[
            pltpu.VMEM((N, CP), f32),
            pltpu.VMEM((N, CP), f32),
            pltpu.VMEM((BT, CP), f32),
        ],
    )(u_pad, m)

    return q[:, :C].reshape(H, W, C)


# speedup vs baseline: 2.0507x; 2.0507x over previous
import math

import jax
import jax.numpy as jnp
from jax.experimental import pallas as pl
from jax.experimental.pallas import tpu as pltpu

H, W, C = 64, 64, 21
N = H * W
THETA_ALPHA, THETA_BETA, THETA_GAMMA = 80.0, 13.0, 3.0
BILATERAL_COMPAT, SPATIAL_COMPAT = 10.0, 3.0
NUM_ITERATIONS = 5

CP = 128
BIG = 1.0e9
BT = 512
NI = N // BT

HALF_LN_BC = 0.5 * math.log(BILATERAL_COMPAT)
HALF_LN_SC = 0.5 * math.log(SPATIAL_COMPAT)


def _rowsum_body(fb_ref, fbT_ref, fs_ref, fsT_ref, ab_ref, as_ref,
                 accb, accs):
    j = pl.program_id(1)

    @pl.when(j == 0)
    def _zero():
        accb[...] = jnp.zeros_like(accb)
        accs[...] = jnp.zeros_like(accs)

    def tile_exp(f_ref, fT_ref):
        fi = f_ref[...]
        fT = fT_ref[...]
        sqi = jnp.sum(fi * fi, axis=1, keepdims=True)
        sqj = jnp.sum(fT * fT, axis=0, keepdims=True)
        cross = jnp.dot(fi, fT, preferred_element_type=jnp.float32)
        return jnp.exp((cross - 0.5 * sqj) - 0.5 * sqi), sqi

    eb, sqib = tile_exp(fb_ref, fbT_ref)
    es, sqis = tile_exp(fs_ref, fsT_ref)
    accb[...] += jnp.sum(eb, axis=1, keepdims=True)
    accs[...] += jnp.sum(es, axis=1, keepdims=True)

    @pl.when(j == NI - 1)
    def _finish():
        ab_ref[...] = (HALF_LN_BC - 0.5 * sqib
                       - jnp.log(jnp.sqrt(accb[...]) + 1e-20))
        as_ref[...] = (HALF_LN_SC - 0.5 * sqis
                       - jnp.log(jnp.sqrt(accs[...]) + 1e-20))


def _build_m_body(fb_ref, fbT_ref, fs_ref, fsT_ref,
                  ab_ref, abT_ref, as_ref, asT_ref, m_ref):
    def tile(f_ref, fT_ref, a_ref, aT_ref):
        cross = jnp.dot(f_ref[...], fT_ref[...],
                        preferred_element_type=jnp.float32)
        return jnp.exp((cross + aT_ref[...]) + a_ref[...])

    mb = tile(fb_ref, fbT_ref, ab_ref, abT_ref)
    ms = tile(fs_ref, fsT_ref, as_ref, asT_ref)
    m_ref[...] = (mb + ms).astype(jnp.bfloat16)


def _softmax(x):
    m = jnp.max(x, axis=-1, keepdims=True)
    e = jnp.exp(x - m)
    return e / jnp.sum(e, axis=-1, keepdims=True)


def _iterate_body(u_ref, m_ref, out_ref, qa, qb):
    t = pl.program_id(0)
    i = pl.program_id(1)

    @pl.when((t == 0) & (i == 0))
    def _init():
        qa[...] = _softmax(-u_ref[...]).astype(jnp.bfloat16)

    read_a = (t % 2) == 0
    m = m_ref[...]

    def step(qsrc, qdst):
        acc = jnp.dot(m, qsrc[...], preferred_element_type=jnp.float32)
        logits = acc - u_ref[pl.ds(i * BT, BT), :]
        qnew = _softmax(logits)
        qdst[pl.ds(i * BT, BT), :] = qnew.astype(jnp.bfloat16)

        @pl.when(t == NUM_ITERATIONS - 1)
        def _out():
            out_ref[pl.ds(i * BT, BT), :] = qnew

    @pl.when(read_a)
    def _step_a():
        step(qa, qb)

    @pl.when(jnp.logical_not(read_a))
    def _step_b():
        step(qb, qa)


@jax.jit
def kernel(unary, image):
    f32 = jnp.float32
    ys, xs = jnp.meshgrid(jnp.arange(H, dtype=f32),
                          jnp.arange(W, dtype=f32), indexing="ij")
    zeros1 = jnp.zeros((N, 1), f32)
    fb = jnp.concatenate(
        [(xs / THETA_ALPHA).reshape(N, 1), (ys / THETA_ALPHA).reshape(N, 1),
         (image / THETA_BETA).reshape(N, 3), zeros1, zeros1, zeros1], axis=1)
    fs = jnp.concatenate(
        [(xs / THETA_GAMMA).reshape(N, 1), (ys / THETA_GAMMA).reshape(N, 1)]
        + [zeros1] * 6, axis=1)
    fbT = fb.T
    fsT = fs.T

    ab, a_s = pl.pallas_call(
        _rowsum_body,
        grid=(NI, NI),
        in_specs=[
            pl.BlockSpec((BT, 8), lambda i, j: (i, 0)),
            pl.BlockSpec((8, BT), lambda i, j: (0, j)),
            pl.BlockSpec((BT, 8), lambda i, j: (i, 0)),
            pl.BlockSpec((8, BT), lambda i, j: (0, j)),
        ],
        out_specs=[
            pl.BlockSpec((BT, 1), lambda i, j: (i, 0)),
            pl.BlockSpec((BT, 1), lambda i, j: (i, 0)),
        ],
        out_shape=[
            jax.ShapeDtypeStruct((N, 1), f32),
            jax.ShapeDtypeStruct((N, 1), f32),
        ],
        scratch_shapes=[
            pltpu.VMEM((BT, 1), f32),
            pltpu.VMEM((BT, 1), f32),
        ],
    )(fb, fbT, fs, fsT)
    abT = ab.reshape(1, N)
    asT = a_s.reshape(1, N)

    m = pl.pallas_call(
        _build_m_body,
        grid=(NI, NI),
        in_specs=[
            pl.BlockSpec((BT, 8), lambda i, j: (i, 0)),
            pl.BlockSpec((8, BT), lambda i, j: (0, j)),
            pl.BlockSpec((BT, 8), lambda i, j: (i, 0)),
            pl.BlockSpec((8, BT), lambda i, j: (0, j)),
            pl.BlockSpec((BT, 1), lambda i, j: (i, 0)),
            pl.BlockSpec((1, BT), lambda i, j: (0, j)),
            pl.BlockSpec((BT, 1), lambda i, j: (i, 0)),
            pl.BlockSpec((1, BT), lambda i, j: (0, j)),
        ],
        out_specs=pl.BlockSpec((BT, BT), lambda i, j: (i, j)),
        out_shape=jax.ShapeDtypeStruct((N, N), jnp.bfloat16),
    )(fb, fbT, fs, fsT, ab, abT, a_s, asT)

    u = unary.reshape(N, C)
    u_pad = jnp.full((N, CP), BIG, f32).at[:, :C].set(u)

    q = pl.pallas_call(
        _iterate_body,
        grid=(NUM_ITERATIONS, NI),
        in_specs=[
            pl.BlockSpec((N, CP), lambda t, i: (0, 0)),
            pl.BlockSpec((BT, N), lambda t, i: (i, 0)),
        ],
        out_specs=pl.BlockSpec((N, CP), lambda t, i: (0, 0)),
        out_shape=jax.ShapeDtypeStruct((N, CP), f32),
        scratch_shapes=[
            pltpu.VMEM((N, CP), jnp.bfloat16),
            pltpu.VMEM((N, CP), jnp.bfloat16),
        ],
    )(u_pad, m)

    return q[:, :C].reshape(H, W, C)
